# baseline (speedup 1.0000x reference)
import functools

import jax
import jax.numpy as jnp
from jax import lax
from jax.experimental import pallas as pl
from jax.experimental.pallas import tpu as pltpu
from jax.experimental.pallas import tpu_sc as plsc

_NC = 2
_NS = 16
_NW = _NC * _NS
_L = 16

_BB = 128
_FPAD = 32
_PACK = 4


@functools.lru_cache(maxsize=None)
def _build_repack(rows, dim):
    slab = 320
    nslab = rows // slab
    assert nslab * slab == rows
    _c = -(-nslab // _NW)
    nstep = _c + (_c % 2)
    mesh = plsc.VectorSubcoreMesh(core_axis_name="c", subcore_axis_name="s")

    @functools.partial(
        pl.kernel,
        out_type=jax.ShapeDtypeStruct((rows // _PACK, _PACK * dim),
                                      jnp.float32),
        mesh=mesh,
        scratch_types=[
            pltpu.VMEM((2, slab, dim), jnp.float32),
            pltpu.VMEM((2, slab // _PACK, _PACK * dim), jnp.float32),
            pltpu.SemaphoreType.DMA,
            pltpu.SemaphoreType.DMA,
            pltpu.SemaphoreType.DMA,
            pltpu.SemaphoreType.DMA,
        ],
        compiler_params=pltpu.CompilerParams(
            use_tc_tiling_on_sc=True, needs_layout_passes=False),
    )
    def repack_kernel(t_hbm, out_hbm, in_v, out_v,
                      sem_in0, sem_in1, sem_o0, sem_o1):
        sem_in = (sem_in0, sem_in1)
        sem_o = (sem_o0, sem_o1)
        wid = lax.axis_index("s") * _NC + lax.axis_index("c")

        def live(j):
            return wid + _NW * j < nslab

        def issue_in(j, p):
            g = wid + _NW * j
            r0 = pl.multiple_of(g * slab, 8)
            pltpu.async_copy(t_hbm.at[pl.ds(r0, slab)], in_v.at[p],
                             sem_in[p])

        def wait_in(p):
            pltpu.make_async_copy(t_hbm.at[pl.ds(0, slab)], in_v.at[p],
                                  sem_in[p]).wait()

        def wait_out(p):
            pltpu.make_async_copy(out_v.at[p],
                                  out_hbm.at[pl.ds(0, slab // _PACK)],
                                  sem_o[p]).wait()

        def repack(p):
            def quad(q, carry):
                for h in range(_PACK * dim // _L):
                    r = _PACK * q + (h * _L) // dim
                    c = (h * _L) % dim
                    out_v[p, q, pl.ds(h * _L, _L)] = (
                        in_v[p, r, pl.ds(c, _L)])
                return carry
            lax.fori_loop(0, slab // _PACK, quad, 0)

        issue_in(0, 0)
        issue_in(1, 1)

        def body(t, carry):
            for p in (0, 1):
                j = 2 * t + p

                @pl.when(live(j))
                def _():
                    wait_in(p)
                    @pl.when(t >= 1)
                    def _():
                        wait_out(p)
                    repack(p)
                    g = wid + _NW * j
                    q0 = pl.multiple_of(g * (slab // _PACK), 8)
                    pltpu.async_copy(out_v.at[p],
                                     out_hbm.at[pl.ds(q0, slab // _PACK)],
                                     sem_o[p])

                @pl.when(live(j + 2))
                def _():
                    issue_in(j + 2, p)
            return carry

        lax.fori_loop(0, nstep // 2, body, 0)
        wait_out(0)
        wait_out(1)

    return repack_kernel


@functools.lru_cache(maxsize=None)
def _build(batch, fields, dim, rows):
    nblocks = batch // _BB
    bpw = nblocks // _NW
    nstream = bpw * fields
    assert nstream % 2 == 0
    nj8 = fields * dim // 8
    mesh = plsc.VectorSubcoreMesh(core_axis_name="c", subcore_axis_name="s")

    @functools.partial(
        pl.kernel,
        out_type=jax.ShapeDtypeStruct((nj8, nblocks, 8, _BB), jnp.float32),
        mesh=mesh,
        scratch_types=[
            pltpu.VMEM((bpw, _FPAD, _BB), jnp.int32),
            pltpu.VMEM((bpw, _FPAD, _BB), jnp.int32),
            pltpu.VMEM((_FPAD, _L), jnp.int32),
            pltpu.VMEM((2, _BB, _BB), jnp.float32),
            pltpu.VMEM((2, dim // 8, 1, 1, 8, _BB), jnp.float32),
            pltpu.SemaphoreType.DMA,
            pltpu.SemaphoreType.DMA,
            pltpu.SemaphoreType.DMA,
            pltpu.SemaphoreType.DMA,
            pltpu.SemaphoreType.DMA,
        ],
        compiler_params=pltpu.CompilerParams(
            use_tc_tiling_on_sc=True, needs_layout_passes=False),
    )
    def gather_kernel(idx_hbm, off_hbm, t4_hbm, out_hbm,
                      slab_s, slab_r, off_v, gbuf, tbuf,
                      sem_in, sem_g0, sem_g1, sem_t0, sem_t1):
        sem_g = (sem_g0, sem_g1)
        sem_t = (sem_t0, sem_t1)
        wid = lax.axis_index("s") * _NC + lax.axis_index("c")
        blk0 = wid * bpw
        lanes = lax.broadcasted_iota(jnp.int32, (_L,), 0)

        pltpu.sync_copy(off_hbm, off_v)
        pltpu.async_copy(idx_hbm.at[pl.ds(pl.multiple_of(blk0, bpw), bpw)],
                         slab_s, sem_in).wait()

        def shift(i, carry):
            blk = i // fields
            f = lax.rem(i, fields)
            off16 = off_v[f, pl.ds(0, _L)]
            for k in range(_BB // _L):
                s = pl.ds(k * _L, _L)
                v = slab_s[blk, f, s] + off16
                slab_s[blk, f, s] = v
                slab_r[blk, f, s] = lax.shift_right_logical(v, 2)
            return carry
        lax.fori_loop(0, nstream, shift, 0)

        def fire(i, q):
            blk = i // fields
            f = lax.rem(i, fields)
            pltpu.async_copy(t4_hbm.at[slab_r.at[blk, f]], gbuf.at[q],
                             sem_g[q])

        def wait_g(q):
            pltpu.make_async_copy(t4_hbm.at[slab_r.at[0, 0]], gbuf.at[q],
                                  sem_g[q]).wait()

        def wait_tiles(q):
            for t in range(dim // 8):
                pltpu.make_async_copy(
                    tbuf.at[q, t], out_hbm.at[pl.ds(0, 1), pl.ds(0, 1)],
                    sem_t[q]).wait()

        def process(i, q):
            blk = i // fields
            f = lax.rem(i, fields)
            for k in range(_BB // _L):
                idx16 = slab_s[blk, f, pl.ds(k * _L, _L)]
                m16 = (idx16 & (_PACK - 1)) * dim
                rowk = lanes + (k * _L)
                for d in range(dim):
                    x = plsc.load_gather(gbuf.at[q], [rowk, m16 + d])
                    tbuf[q, d // 8, 0, 0, d % 8, pl.ds(k * _L, _L)] = x
            bbg = blk0 + blk
            for t in range(dim // 8):
                j8 = f * (dim // 8) + t
                pltpu.async_copy(
                    tbuf.at[q, t],
                    out_hbm.at[pl.ds(j8, 1), pl.ds(bbg, 1)], sem_t[q])

        fire(0, 0)

        def body(u, carry):
            i0 = 2 * u
            fire(i0 + 1, 1)
            wait_g(0)
            @pl.when(u >= 1)
            def _():
                wait_tiles(0)
            process(i0, 0)

            @pl.when(u < (nstream // 2) - 1)
            def _():
                fire(i0 + 2, 0)
            wait_g(1)
            @pl.when(u >= 1)
            def _():
                wait_tiles(1)
            process(i0 + 1, 1)
            return carry

        lax.fori_loop(0, nstream // 2, body, 0)
        wait_tiles(0)
        wait_tiles(1)

    return gather_kernel


def kernel(input, num_classes, table):
    batch, fields = input.shape
    rows, dim = table.shape
    offsets = jnp.concatenate([
        jnp.zeros((1,), dtype=num_classes.dtype),
        jnp.cumsum(num_classes)[:-1],
    ]).astype(jnp.int32)
    off_bc = jnp.zeros((_FPAD, _L), jnp.int32).at[:fields, :].set(
        jnp.broadcast_to(offsets[:, None], (fields, _L)))
    idx_fm = jnp.pad(
        input.T.reshape(fields, batch // _BB, _BB).transpose(1, 0, 2),
        ((0, 0), (0, _FPAD - fields), (0, 0)))
    t4 = _build_repack(rows, dim)(table)
    out4 = _build(batch, fields, dim, rows)(idx_fm, off_bc, t4)
    return out4.transpose(1, 3, 0, 2).reshape(batch, fields * dim)


# speedup vs baseline: 1.2159x; 1.2159x over previous
import functools

import jax
import jax.numpy as jnp
from jax import lax
from jax.experimental import pallas as pl
from jax.experimental.pallas import tpu as pltpu
from jax.experimental.pallas import tpu_sc as plsc

_NC = 2
_NS = 16
_NW = _NC * _NS
_L = 16

_CHUNK = 1664
_IPS = 128
_K = _CHUNK // _IPS


@functools.lru_cache(maxsize=None)
def _build(total, dim):
    assert total % (_NW * _CHUNK) == 0
    per_w = total // _NW
    nchunk = per_w // _CHUNK
    assert nchunk % 2 == 0
    mesh = plsc.VectorSubcoreMesh(core_axis_name="c", subcore_axis_name="s")

    @functools.partial(
        pl.kernel,
        out_type=jax.ShapeDtypeStruct((total, dim), jnp.float32),
        mesh=mesh,
        scratch_types=[
            pltpu.VMEM((2, _K, _IPS), jnp.int32),
            pltpu.VMEM((2, _K, _IPS), jnp.int32),
            pltpu.VMEM((2, _CHUNK, dim), jnp.float32),
            pltpu.SemaphoreType.DMA,
            pltpu.SemaphoreType.DMA,
            pltpu.SemaphoreType.DMA,
            pltpu.SemaphoreType.DMA,
            pltpu.SemaphoreType.DMA,
            pltpu.SemaphoreType.DMA,
        ],
        compiler_params=pltpu.CompilerParams(use_tc_tiling_on_sc=False),
    )
    def gather_kernel(idx_hbm, off_hbm, table_hbm, out_hbm,
                      idx_v, off_v, rows_v,
                      sem_in0, sem_in1, sem_g0, sem_g1, sem_wb0, sem_wb1):
        sem_in = (sem_in0, sem_in1)
        sem_g = (sem_g0, sem_g1)
        sem_wb = (sem_wb0, sem_wb1)
        wid = lax.axis_index("s") * _NC + lax.axis_index("c")
        cbase = wid * nchunk

        def issue_in(gid, b):
            pltpu.async_copy(idx_hbm.at[gid], idx_v.at[b], sem_in[b])
            pltpu.async_copy(off_hbm.at[gid], off_v.at[b], sem_in[b])

        def wait_in(b):
            pltpu.make_async_copy(idx_hbm.at[0], idx_v.at[b], sem_in[b]).wait()
            pltpu.make_async_copy(off_hbm.at[0], off_v.at[b], sem_in[b]).wait()

        def adds(b):
            for j in range(_K):
                for i in range(_IPS // _L):
                    s = pl.ds(i * _L, _L)
                    idx_v[b, j, s] = idx_v[b, j, s] + off_v[b, j, s]

        def fire(b):
            for j in range(_K):
                pltpu.async_copy(table_hbm.at[idx_v.at[b, j]],
                                 rows_v.at[b, pl.ds(j * _IPS, _IPS)],
                                 sem_g[b])

        def wait_gathers(b):
            for j in range(_K):
                pltpu.make_async_copy(
                    table_hbm.at[idx_v.at[b, j]],
                    rows_v.at[b, pl.ds(j * _IPS, _IPS)], sem_g[b]).wait()

        def issue_wb(gid, b):
            cb = pl.multiple_of(gid * _CHUNK, 8)
            pltpu.async_copy(rows_v.at[b], out_hbm.at[pl.ds(cb, _CHUNK)],
                             sem_wb[b])

        def wait_wb(b):
            pltpu.make_async_copy(rows_v.at[b], out_hbm.at[pl.ds(0, _CHUNK)],
                                  sem_wb[b]).wait()

        issue_in(cbase, 0)
        issue_in(cbase + 1, 1)
        wait_in(0)
        adds(0)

        def body(t, carry):
            for p, b in ((0, 0), (1, 1)):
                gid = cbase + 2 * t + p
                @pl.when(t >= 1)
                def _():
                    wait_wb(b)
                fire(b)

                def prep():
                    wait_in(1 - b)
                    adds(1 - b)
                if p == 0:
                    prep()
                else:
                    pl.when(t < (nchunk // 2) - 1)(prep)
                wait_gathers(b)
                issue_wb(gid, b)
                @pl.when(t < (nchunk // 2) - 1)
                def _():
                    issue_in(gid + 2, b)
            return carry

        lax.fori_loop(0, nchunk // 2, body, 0)
        wait_wb(0)
        wait_wb(1)

    return gather_kernel


def kernel(input, num_classes, table):
    batch, fields = input.shape
    dim = table.shape[1]
    total = batch * fields
    offsets = jnp.concatenate([
        jnp.zeros((1,), dtype=num_classes.dtype),
        jnp.cumsum(num_classes)[:-1],
    ]).astype(jnp.int32)
    nblk = total // _CHUNK
    idx3 = input.reshape(nblk, _K, _IPS)
    off3 = jnp.broadcast_to(offsets, (batch, fields)).reshape(
        nblk, _K, _IPS)
    out = _build(total, dim)(idx3, off3, table)
    return out.reshape(batch, fields * dim)
